# initial kernel scaffold (unmeasured)
import jax
import jax.numpy as jnp
from jax import lax
from jax.experimental import pallas as pl
from jax.experimental.pallas import tpu as pltpu

N_DEV = 32
M_BLK = 128
K = 4096
N = 8192
TN = 1024
NT = N // TN


def kernel(x, w_mat):
    m_total, k_shard = x.shape
    assert (m_total, k_shard) == (N_DEV * M_BLK, M_BLK)
    assert w_mat.shape == (K, N)

    def body(x_ref, w_hbm, out_ref,
             xg_ref, wtile_ref, amax_ref,
             send_sems, recv_sems, own_sem,
             amax_send_sems, amax_recv_sems, wdma_sems):
        me = lax.axis_index("i")

        for t in range(1, N_DEV):
            dst = (me + t) % N_DEV
            rdma = pltpu.make_async_remote_copy(
                src_ref=x_ref.at[pl.ds(dst * M_BLK, M_BLK), :],
                dst_ref=xg_ref.at[:, pl.ds(me * M_BLK, M_BLK)],
                send_sem=send_sems.at[dst],
                recv_sem=recv_sems.at[me],
                device_id=(dst,),
                device_id_type=pl.DeviceIdType.MESH,
            )
            rdma.start()

        own_cp = pltpu.make_async_copy(
            x_ref.at[pl.ds(me * M_BLK, M_BLK), :],
            xg_ref.at[:, pl.ds(me * M_BLK, M_BLK)],
            own_sem,
        )
        own_cp.start()

        w_cp0 = pltpu.make_async_copy(
            w_hbm.at[:, pl.ds(0, TN)], wtile_ref.at[0], wdma_sems.at[0]
        )
        w_cp0.start()

        for j in range(N_DEV):
            @pl.when(j != me)
            def _():
                recv = pltpu.make_async_remote_copy(
                    src_ref=x_ref.at[pl.ds(0, M_BLK), :],
                    dst_ref=xg_ref.at[:, pl.ds(j * M_BLK, M_BLK)],
                    send_sem=send_sems.at[j],
                    recv_sem=recv_sems.at[j],
                    device_id=(j,),
                    device_id_type=pl.DeviceIdType.MESH,
                )
                recv.wait_recv()

        own_cp.wait()

        for t in range(1, N_DEV):
            dst = (me + t) % N_DEV
            snd = pltpu.make_async_remote_copy(
                src_ref=x_ref.at[pl.ds(0, M_BLK), :],
                dst_ref=xg_ref.at[:, pl.ds(0, M_BLK)],
                send_sem=send_sems.at[dst],
                recv_sem=recv_sems.at[me],
                device_id=(dst,),
                device_id_type=pl.DeviceIdType.MESH,
            )
            snd.wait_send()

        local_amax = jnp.float32(0.0)
        for it in range(NT):
            slot = it % 2
            if it + 1 < NT:
                nxt = pltpu.make_async_copy(
                    w_hbm.at[:, pl.ds((it + 1) * TN, TN)],
                    wtile_ref.at[(it + 1) % 2],
                    wdma_sems.at[(it + 1) % 2],
                )
                nxt.start()
            cur = pltpu.make_async_copy(
                w_hbm.at[:, pl.ds(it * TN, TN)],
                wtile_ref.at[slot],
                wdma_sems.at[slot],
            )
            cur.wait()
            ytile = jnp.dot(
                xg_ref[...], wtile_ref[slot],
                preferred_element_type=jnp.float32,
            )
            out_ref[:, it * TN:(it + 1) * TN] = ytile
            local_amax = jnp.maximum(local_amax, jnp.max(jnp.abs(ytile)))

        amax_ref[pl.ds(me, 1)] = jnp.full((1, 8, 128), local_amax, jnp.float32)
        for t in range(1, N_DEV):
            dst = (me + t) % N_DEV
            rdma = pltpu.make_async_remote_copy(
                src_ref=amax_ref.at[me],
                dst_ref=amax_ref.at[me],
                send_sem=amax_send_sems.at[dst],
                recv_sem=amax_recv_sems.at[me],
                device_id=(dst,),
                device_id_type=pl.DeviceIdType.MESH,
            )
            rdma.start()
        for j in range(N_DEV):
            @pl.when(j != me)
            def _():
                recv = pltpu.make_async_remote_copy(
                    src_ref=amax_ref.at[j],
                    dst_ref=amax_ref.at[j],
                    send_sem=amax_send_sems.at[j],
                    recv_sem=amax_recv_sems.at[j],
                    device_id=(j,),
                    device_id_type=pl.DeviceIdType.MESH,
                )
                recv.wait_recv()
        for t in range(1, N_DEV):
            dst = (me + t) % N_DEV
            snd = pltpu.make_async_remote_copy(
                src_ref=amax_ref.at[me],
                dst_ref=amax_ref.at[me],
                send_sem=amax_send_sems.at[dst],
                recv_sem=amax_recv_sems.at[me],
                device_id=(dst,),
                device_id_type=pl.DeviceIdType.MESH,
            )
            snd.wait_send()

        gmax = jnp.max(amax_ref[...])
        scale = gmax / 127.0

        y = out_ref[...]
        q = jnp.clip(jnp.round(y / scale), -127.0, 127.0)
        out_ref[...] = q * scale

    return pl.pallas_call(
        body,
        out_shape=jax.ShapeDtypeStruct((M_BLK, N), jnp.float32),
        in_specs=[
            pl.BlockSpec(memory_space=pltpu.VMEM),
            pl.BlockSpec(memory_space=pltpu.ANY),
        ],
        out_specs=pl.BlockSpec(memory_space=pltpu.VMEM),
        scratch_shapes=[
            pltpu.VMEM((M_BLK, K), jnp.bfloat16),
            pltpu.VMEM((2, K, TN), jnp.bfloat16),
            pltpu.VMEM((N_DEV, 8, 128), jnp.float32),
            pltpu.SemaphoreType.DMA((N_DEV,)),
            pltpu.SemaphoreType.DMA((N_DEV,)),
            pltpu.SemaphoreType.DMA,
            pltpu.SemaphoreType.DMA((N_DEV,)),
            pltpu.SemaphoreType.DMA((N_DEV,)),
            pltpu.SemaphoreType.DMA((2,)),
        ],
        compiler_params=pltpu.CompilerParams(collective_id=0),
    )(x, w_mat)


# baseline (device time: 77416 ns/iter reference)
import jax
import jax.numpy as jnp
from jax import lax
from jax.experimental import pallas as pl
from jax.experimental.pallas import tpu as pltpu

N_DEV = 32
M_BLK = 128
K = 4096
N = 8192
TN = 512
NT = N // TN


def kernel(x, w_mat):
    m_total, k_shard = x.shape
    assert (m_total, k_shard) == (N_DEV * M_BLK, M_BLK)
    assert w_mat.shape == (K, N)

    def body(x_ref, w_hbm, out_ref,
             x16_ref, xg_ref, wf32_ref, w16_ref, amax_ref,
             send_sems, recv_sems, own_sem,
             amax_send_sems, amax_recv_sems, wdma_sems):
        me = lax.axis_index("i")

        x16_ref[...] = x_ref[...].astype(jnp.bfloat16)

        for t in range(1, N_DEV):
            dst = (me + t) % N_DEV
            rdma = pltpu.make_async_remote_copy(
                src_ref=x16_ref.at[pl.ds(dst * M_BLK, M_BLK), :],
                dst_ref=xg_ref.at[:, pl.ds(me * M_BLK, M_BLK)],
                send_sem=send_sems.at[dst],
                recv_sem=recv_sems.at[me],
                device_id=(dst,),
                device_id_type=pl.DeviceIdType.MESH,
            )
            rdma.start()

        own_cp = pltpu.make_async_copy(
            x16_ref.at[pl.ds(me * M_BLK, M_BLK), :],
            xg_ref.at[:, pl.ds(me * M_BLK, M_BLK)],
            own_sem,
        )
        own_cp.start()

        w_cp0 = pltpu.make_async_copy(
            w_hbm.at[:, pl.ds(0, TN)], wf32_ref.at[0], wdma_sems.at[0]
        )
        w_cp0.start()

        for j in range(N_DEV):
            @pl.when(j != me)
            def _():
                recv = pltpu.make_async_remote_copy(
                    src_ref=x16_ref.at[pl.ds(0, M_BLK), :],
                    dst_ref=xg_ref.at[:, pl.ds(j * M_BLK, M_BLK)],
                    send_sem=send_sems.at[j],
                    recv_sem=recv_sems.at[j],
                    device_id=(j,),
                    device_id_type=pl.DeviceIdType.MESH,
                )
                recv.wait_recv()

        own_cp.wait()

        for t in range(1, N_DEV):
            dst = (me + t) % N_DEV
            snd = pltpu.make_async_remote_copy(
                src_ref=x16_ref.at[pl.ds(0, M_BLK), :],
                dst_ref=xg_ref.at[:, pl.ds(0, M_BLK)],
                send_sem=send_sems.at[dst],
                recv_sem=recv_sems.at[me],
                device_id=(dst,),
                device_id_type=pl.DeviceIdType.MESH,
            )
            snd.wait_send()

        local_amax = jnp.float32(0.0)
        for it in range(NT):
            slot = it % 2
            if it + 1 < NT:
                nxt = pltpu.make_async_copy(
                    w_hbm.at[:, pl.ds((it + 1) * TN, TN)],
                    wf32_ref.at[(it + 1) % 2],
                    wdma_sems.at[(it + 1) % 2],
                )
                nxt.start()
            cur = pltpu.make_async_copy(
                w_hbm.at[:, pl.ds(it * TN, TN)],
                wf32_ref.at[slot],
                wdma_sems.at[slot],
            )
            cur.wait()
            w16_ref[...] = wf32_ref[slot].astype(jnp.bfloat16)
            ytile = jnp.dot(
                xg_ref[...], w16_ref[...],
                preferred_element_type=jnp.float32,
            )
            out_ref[:, it * TN:(it + 1) * TN] = ytile
            local_amax = jnp.maximum(local_amax, jnp.max(jnp.abs(ytile)))

        amax_ref[pl.ds(me, 1)] = jnp.full((1, 8, 128), local_amax, jnp.float32)
        for t in range(1, N_DEV):
            dst = (me + t) % N_DEV
            rdma = pltpu.make_async_remote_copy(
                src_ref=amax_ref.at[me],
                dst_ref=amax_ref.at[me],
                send_sem=amax_send_sems.at[dst],
                recv_sem=amax_recv_sems.at[me],
                device_id=(dst,),
                device_id_type=pl.DeviceIdType.MESH,
            )
            rdma.start()
        for j in range(N_DEV):
            @pl.when(j != me)
            def _():
                recv = pltpu.make_async_remote_copy(
                    src_ref=amax_ref.at[j],
                    dst_ref=amax_ref.at[j],
                    send_sem=amax_send_sems.at[j],
                    recv_sem=amax_recv_sems.at[j],
                    device_id=(j,),
                    device_id_type=pl.DeviceIdType.MESH,
                )
                recv.wait_recv()
        for t in range(1, N_DEV):
            dst = (me + t) % N_DEV
            snd = pltpu.make_async_remote_copy(
                src_ref=amax_ref.at[me],
                dst_ref=amax_ref.at[me],
                send_sem=amax_send_sems.at[dst],
                recv_sem=amax_recv_sems.at[me],
                device_id=(dst,),
                device_id_type=pl.DeviceIdType.MESH,
            )
            snd.wait_send()

        gmax = jnp.max(amax_ref[...])
        scale = gmax / 127.0

        y = out_ref[...]
        q = jnp.clip(jnp.round(y / scale), -127.0, 127.0)
        out_ref[...] = q * scale

    return pl.pallas_call(
        body,
        out_shape=jax.ShapeDtypeStruct((M_BLK, N), jnp.float32),
        in_specs=[
            pl.BlockSpec(memory_space=pltpu.VMEM),
            pl.BlockSpec(memory_space=pl.ANY),
        ],
        out_specs=pl.BlockSpec(memory_space=pltpu.VMEM),
        scratch_shapes=[
            pltpu.VMEM((N_DEV * M_BLK, M_BLK), jnp.bfloat16),
            pltpu.VMEM((M_BLK, K), jnp.bfloat16),
            pltpu.VMEM((2, K, TN), jnp.float32),
            pltpu.VMEM((K, TN), jnp.bfloat16),
            pltpu.VMEM((N_DEV, 8, 128), jnp.float32),
            pltpu.SemaphoreType.DMA((N_DEV,)),
            pltpu.SemaphoreType.DMA((N_DEV,)),
            pltpu.SemaphoreType.DMA,
            pltpu.SemaphoreType.DMA((N_DEV,)),
            pltpu.SemaphoreType.DMA((N_DEV,)),
            pltpu.SemaphoreType.DMA((2,)),
        ],
    )(x, w_mat)
